# baseline (speedup 1.0000x reference)
import functools
from itertools import combinations

import jax
import jax.numpy as jnp
from jax import lax
from jax.experimental import pallas as pl
from jax.experimental.pallas import tpu as pltpu
from jax.experimental.pallas import tpu_sc as plsc

B, F, D = 16384, 5, 128
L = 16
NDC = D // L
ROW = F * D

_TRIPLES = list(combinations(range(F), 3))
_PAIRS = sorted({(i, j) for (i, j, _k) in _TRIPLES})

_info = plsc.get_sparse_core_info()
NC, NS = _info.num_cores, _info.num_subcores
NW = NC * NS
RPW = B // NW
C = 64
NG = RPW // C
GPC = C // L


@functools.partial(
    pl.kernel,
    mesh=plsc.VectorSubcoreMesh(core_axis_name="c", subcore_axis_name="s"),
    compiler_params=pltpu.CompilerParams(needs_layout_passes=False),
    out_type=jax.ShapeDtypeStruct((B,), jnp.float32),
    scratch_types=[
        pltpu.VMEM((C, ROW), jnp.float32),
        pltpu.VMEM((C, ROW), jnp.float32),
        pltpu.VMEM((len(_TRIPLES), L), jnp.float32),
        pltpu.VMEM((L * L,), jnp.float32),
        pltpu.VMEM((RPW,), jnp.float32),
        pltpu.SemaphoreType.DMA,
        pltpu.SemaphoreType.DMA,
    ],
)
def _sc_triple_sum(x_hbm, wb_hbm, out_hbm, buf0, buf1, wb_v, tbuf, out_v,
                   sem0, sem1):
    wid = lax.axis_index("s") * NC + lax.axis_index("c")
    base = wid * RPW

    pltpu.sync_copy(wb_hbm, wb_v)
    wv = [wb_v[t] for t in range(len(_TRIPLES))]
    col_idx = lax.iota(jnp.int32, L) * L

    bufs, sems = (buf0, buf1), (sem0, sem1)
    copies = [None, None]
    copies[0] = pltpu.async_copy(
        x_hbm.at[pl.ds(base, C)], bufs[0], sems[0])

    for g in range(NG):
        if g + 1 < NG:
            nb = (g + 1) % 2
            copies[nb] = pltpu.async_copy(
                x_hbm.at[pl.ds(base + (g + 1) * C, C)], bufs[nb], sems[nb])
        copies[g % 2].wait()
        buf = bufs[g % 2]

        def group_body(g2, _, buf=buf, g=g):
            @plsc.parallel_loop(0, L, unroll=2)
            def row_body(rr, buf=buf, g2=g2):
                r = g2 * L + rr
                acc = None
                for c in range(NDC):
                    xs = [buf[r, pl.ds(f * D + c * L, L)] for f in range(F)]
                    pp = {ij: xs[ij[0]] * xs[ij[1]] for ij in _PAIRS}
                    for k in range(2, F):
                        inner = None
                        for t, (i, j, kk) in enumerate(_TRIPLES):
                            if kk != k:
                                continue
                            term = wv[t] * pp[(i, j)]
                            inner = term if inner is None else inner + term
                        contrib = xs[k] * inner
                        acc = contrib if acc is None else acc + contrib
                tbuf[pl.ds(rr * L, L)] = acc
            total = None
            for c in range(L):
                col = plsc.load_gather(tbuf, [col_idx + c])
                total = col if total is None else total + col
            out_v[pl.ds(g * C + g2 * L, L)] = total
            return 0

        lax.fori_loop(0, GPC, group_body, 0)

    pltpu.sync_copy(out_v, out_hbm.at[pl.ds(base, RPW)])


@jax.jit
def kernel(x, w):
    xr = x.reshape(B, ROW)
    wb = jnp.broadcast_to(w[:, None], (len(_TRIPLES), L))
    out = _sc_triple_sum(xr, wb)
    return out.reshape(B, 1)


# speedup vs baseline: 1.2565x; 1.2565x over previous
import functools
from itertools import combinations

import jax
import jax.numpy as jnp
from jax import lax
from jax.experimental import pallas as pl
from jax.experimental.pallas import tpu as pltpu
from jax.experimental.pallas import tpu_sc as plsc

B, F, D = 16384, 5, 128
L = 16
NDC = D // L
ROW = F * D

_TRIPLES = list(combinations(range(F), 3))
_PAIRS = sorted({(i, j) for (i, j, _k) in _TRIPLES})

_info = plsc.get_sparse_core_info()
NC, NS = _info.num_cores, _info.num_subcores
NW = NC * NS
RPW = B // NW
C = 64
NG = RPW // C
GPC = C // L


@functools.partial(
    pl.kernel,
    mesh=plsc.VectorSubcoreMesh(core_axis_name="c", subcore_axis_name="s"),
    compiler_params=pltpu.CompilerParams(needs_layout_passes=False),
    out_type=jax.ShapeDtypeStruct((B,), jnp.float32),
    scratch_types=[
        pltpu.VMEM((C, ROW), jnp.float32),
        pltpu.VMEM((C, ROW), jnp.float32),
        pltpu.VMEM((len(_TRIPLES), L), jnp.float32),
        pltpu.VMEM((L * L,), jnp.float32),
        pltpu.VMEM((RPW,), jnp.float32),
        pltpu.SemaphoreType.DMA,
        pltpu.SemaphoreType.DMA,
    ],
)
def _sc_triple_sum(x_hbm, wb_hbm, out_hbm, buf0, buf1, wb_v, tbuf, out_v,
                   sem0, sem1):
    wid = lax.axis_index("s") * NC + lax.axis_index("c")
    base = wid * RPW

    pltpu.sync_copy(wb_hbm, wb_v)
    wv = [wb_v[t] for t in range(len(_TRIPLES))]
    col_idx = lax.iota(jnp.int32, L) * L

    bufs, sems = (buf0, buf1), (sem0, sem1)
    copies = [None, None]
    copies[0] = pltpu.async_copy(
        x_hbm.at[pl.ds(base, C)], bufs[0], sems[0])

    for g in range(NG):
        if g + 1 < NG:
            nb = (g + 1) % 2
            copies[nb] = pltpu.async_copy(
                x_hbm.at[pl.ds(base + (g + 1) * C, C)], bufs[nb], sems[nb])
        copies[g % 2].wait()
        buf = bufs[g % 2]

        def group_body(g2, _, buf=buf, g=g):
            @plsc.parallel_loop(0, L, unroll=1)
            def row_body(rr, buf=buf, g2=g2):
                r = g2 * L + rr
                acc = None
                for c in range(NDC):
                    xs = [buf[r, pl.ds(f * D + c * L, L)] for f in range(F)]
                    pp = {ij: xs[ij[0]] * xs[ij[1]] for ij in _PAIRS}
                    for k in range(2, F):
                        inner = None
                        for t, (i, j, kk) in enumerate(_TRIPLES):
                            if kk != k:
                                continue
                            term = wv[t] * pp[(i, j)]
                            inner = term if inner is None else inner + term
                        contrib = xs[k] * inner
                        acc = contrib if acc is None else acc + contrib
                tbuf[pl.ds(rr * L, L)] = acc
            total = None
            for c in range(L):
                col = plsc.load_gather(tbuf, [col_idx + c])
                total = col if total is None else total + col
            out_v[pl.ds(g * C + g2 * L, L)] = total
            return 0

        lax.fori_loop(0, GPC, group_body, 0)

    pltpu.sync_copy(out_v, out_hbm.at[pl.ds(base, RPW)])


@jax.jit
def kernel(x, w):
    xr = x.reshape(B, ROW)
    wb = jnp.broadcast_to(w[:, None], (len(_TRIPLES), L))
    out = _sc_triple_sum(xr, wb)
    return out.reshape(B, 1)
